# baseline (speedup 1.0000x reference)
import jax
import jax.numpy as jnp
from jax import lax
from jax.experimental import pallas as pl
from jax.experimental.pallas import tpu as pltpu
from jax.experimental.pallas import tpu_sc as plsc

B = 16384
D = 64
OUT_COLS = 453

NC = 2
NS = 16
NW = NC * NS
B_PER_W = B // NW
CH = 128
N_CHUNKS = B_PER_W // CH
NG = CH // 16

EMB_COLS = (0, 64, 131, 195, 259, 323, 389)
SCAL = (
    (128, 100.0, 50.0),
    (129, 0.3, 0.1),
    (130, 500.0, 200.0),
    (387, 45.0, 23.0),
    (388, 130.0, 58.0),
)


def _body(i0, i1, i2, i3, i4, i5, i6,
          p0, p1, p2, p3, p4, p5, p6,
          sales, gmii, visits, price, ml,
          t0, t1, t2, t3, t4, t5, t6,
          out_hbm,
          iv0, iv1, iv2, iv3, iv4, iv5, iv6,
          pv0, pv1, pv2, pv3, pv4, pv5, pv6,
          sv0, sv1, sv2, sv3, sv4,
          ga, gb, asm, sem_s, sem_a, sem_b):
    idx_hbm = (i0, i1, i2, i3, i4, i5, i6)
    par_hbm = (p0, p1, p2, p3, p4, p5, p6)
    tables = (t0, t1, t2, t3, t4, t5, t6)
    ivs = (iv0, iv1, iv2, iv3, iv4, iv5, iv6)
    pvs = (pv0, pv1, pv2, pv3, pv4, pv5, pv6)
    scal_hbm = (sales, gmii, visits, price, ml)
    svs = (sv0, sv1, sv2, sv3, sv4)
    gbufs = (ga, gb)
    gsems = (sem_a, sem_b)

    wid = lax.axis_index("s") * NC + lax.axis_index("c")
    base = wid * B_PER_W
    lane = lax.iota(jnp.int32, 16)
    rows_g = [lane + g * 16 for g in range(NG)]

    def chunk_body(cc, carry):
        rbase = pl.multiple_of(base + cc * CH, CH)
        hs = []
        for t in range(7):
            hs.append(pltpu.async_copy(
                idx_hbm[t].at[pl.ds(rbase, CH)], ivs[t], sem_s))
            hs.append(pltpu.async_copy(
                par_hbm[t].at[pl.ds(rbase, CH)], pvs[t], sem_s))
        for f in range(5):
            hs.append(pltpu.async_copy(
                scal_hbm[f].at[pl.ds(rbase, CH)], svs[f], sem_s))
        for h in hs:
            h.wait()
        gh = pltpu.async_copy(tables[0].at[ivs[0]], gbufs[0], gsems[0])

        for f, (col, mean, std) in enumerate(SCAL):
            cols = jnp.full((16,), col, jnp.int32)
            inv = 1.0 / std
            for g in range(NG):
                v = svs[f][pl.ds(g * 16, 16)]
                plsc.store_scatter(asm, [rows_g[g], cols], (v - mean) * inv)

        for t in range(7):
            gh.wait()
            if t < 7 - 1:
                gh = pltpu.async_copy(
                    tables[t + 1].at[ivs[t + 1]],
                    gbufs[(t + 1) % 2], gsems[(t + 1) % 2])
            gbuf = gbufs[t % 2]
            par_g = [pvs[t][pl.ds(g * 16, 16)] for g in range(NG)]

            def c_body(c, inner):
                cs = jnp.full((16,), c, jnp.int32)
                dst = cs + EMB_COLS[t]
                for g in range(NG):
                    v = plsc.load_gather(gbuf, [rows_g[g], par_g[g] + cs])
                    plsc.store_scatter(asm, [rows_g[g], dst], v)
                return inner

            lax.fori_loop(0, D, c_body, 0)
        pltpu.sync_copy(asm, out_hbm.at[pl.ds(rbase, CH)])
        return carry

    lax.fori_loop(0, N_CHUNKS, chunk_body, 0)


@jax.jit
def _sc_call(*args):
    mesh = plsc.VectorSubcoreMesh(core_axis_name="c", subcore_axis_name="s")
    return pl.kernel(
        _body,
        out_type=jax.ShapeDtypeStruct((B, OUT_COLS), jnp.float32),
        mesh=mesh,
        compiler_params=pltpu.CompilerParams(needs_layout_passes=False),
        scratch_types=(
            [pltpu.VMEM((CH,), jnp.int32) for _ in range(7)]
            + [pltpu.VMEM((CH,), jnp.int32) for _ in range(7)]
            + [pltpu.VMEM((CH,), jnp.float32) for _ in range(5)]
            + [pltpu.VMEM((CH, 2 * D), jnp.float32),
               pltpu.VMEM((CH, 2 * D), jnp.float32),
               pltpu.VMEM((CH, OUT_COLS), jnp.float32),
               pltpu.SemaphoreType.DMA,
               pltpu.SemaphoreType.DMA,
               pltpu.SemaphoreType.DMA]
        ),
    )(*args)


def kernel(config_id, brand, category, ptype, series, gender, attributes,
           sales, gmii, visits, price, ml,
           table_product, table_brand, table_category, table_type,
           table_series, table_gender, table_attr):
    idx = [config_id.astype(jnp.int32), brand.astype(jnp.int32),
           category.astype(jnp.int32), ptype.astype(jnp.int32),
           series.astype(jnp.int32), gender.astype(jnp.int32),
           attributes.astype(jnp.int32)]
    pair_idx = [i >> 1 for i in idx]
    half_off = [(i & 1) << 6 for i in idx]
    tviews = [t.reshape(t.shape[0] // 2, 2 * D)
              for t in (table_product, table_brand, table_category,
                        table_type, table_series, table_gender, table_attr)]
    return _sc_call(*pair_idx, *half_off, sales, gmii, visits, price, ml,
                    *tviews)
